# Initial kernel scaffold; baseline (speedup 1.0000x reference)
import jax
import jax.numpy as jnp
from jax.experimental import pallas as pl


def kernel(x_idx, edge_index, edge_attr, batch, params):
    raise NotImplementedError("write your pallas kernel here")


# speedup vs baseline: 4.4071x; 4.4071x over previous
import functools

import jax
import jax.numpy as jnp
from jax import lax
from jax.experimental import pallas as pl
from jax.experimental.pallas import tpu as pltpu
from jax.experimental.pallas import tpu_sc as plsc

N = 10000
E = 320000
EMB = 128
HID = 128
NG = 16

NW = 32
BPW_EMB = 384
B_EMB = NW * BPW_EMB
ECH = 128
NCH = 80
EPW = ECH * NCH
EPAD = NW * EPW
NP_AGG = 10240
RPS = NP_AGG // 16
BLKL = 512
JMAX = 20
HALO = 8
XP_ROWS = N + 544
CONV_CLAMP = N + HALO


def _sc_embed(emb, idx3):
    mesh = plsc.VectorSubcoreMesh(core_axis_name="c", subcore_axis_name="s")

    @functools.partial(
        pl.kernel,
        mesh=mesh,
        out_type=jax.ShapeDtypeStruct((B_EMB, EMB), jnp.float32),
        scratch_types=[
            pltpu.VMEM((3, 128), jnp.int32),
            pltpu.VMEM((BPW_EMB, EMB), jnp.float32),
            pltpu.SemaphoreType.DMA,
        ],
    )
    def k(emb_hbm, idx_hbm, out_hbm, idx_v, rows_v, sem):
        wid = lax.axis_index("s") * 2 + lax.axis_index("c")
        pltpu.sync_copy(idx_hbm.at[wid], idx_v)
        for ch in range(3):
            pltpu.async_copy(
                emb_hbm.at[idx_v.at[ch]],
                rows_v.at[pl.ds(ch * 128, 128)],
                sem,
            )
        for ch in range(3):
            pltpu.make_async_copy(
                emb_hbm.at[idx_v.at[ch]],
                rows_v.at[pl.ds(ch * 128, 128)],
                sem,
            ).wait()
        pltpu.sync_copy(rows_v, out_hbm.at[pl.ds(wid * BPW_EMB, BPW_EMB)])

    return k(emb, idx3)


def _sc_msgpass(x, src3, dst3, w3):
    mesh = plsc.VectorSubcoreMesh(core_axis_name="c", subcore_axis_name="s")

    @functools.partial(
        pl.kernel,
        mesh=mesh,
        out_type=jax.ShapeDtypeStruct((2, NP_AGG, EMB), jnp.float32),
        scratch_types=[
            pltpu.VMEM((16, ECH), jnp.int32),
            pltpu.VMEM((16, ECH), jnp.int32),
            pltpu.VMEM((16, ECH), jnp.float32),
            pltpu.VMEM((ECH, EMB), jnp.float32),
            pltpu.VMEM((64, EMB), jnp.float32),
            pltpu.VMEM_SHARED((NP_AGG, EMB), jnp.float32),
            pltpu.SemaphoreType.DMA,
        ],
    )
    def k(x_hbm, src_hbm, dst_hbm, w_hbm, out_hbm,
          src_v, dst_v, w_v, rows_v, zbuf, agg_sh, sem):
        c = lax.axis_index("c")
        s = lax.axis_index("s")
        wid = s * 2 + c

        def zrow(i, carry):
            for q in range(EMB // 16):
                zbuf[i, pl.ds(q * 16, 16)] = jnp.zeros((16,), jnp.float32)
            return carry

        lax.fori_loop(0, 64, zrow, 0)
        for r in range(10):
            pltpu.sync_copy(zbuf, agg_sh.at[pl.ds(s * RPS + r * 64, 64)])
        plsc.subcore_barrier()

        def refill(rb, carry):
            wsl = pl.ds(rb * 16, 16)
            pltpu.sync_copy(src_hbm.at[wid].at[wsl], src_v)
            pltpu.sync_copy(dst_hbm.at[wid].at[wsl], dst_v)
            pltpu.sync_copy(w_hbm.at[wid].at[wsl], w_v)

            def chunk(i, c3):
                pltpu.async_copy(x_hbm.at[src_v.at[i]], rows_v, sem).wait()

                def edge16(t, c2):
                    w16 = w_v[i, pl.ds(t * 16, 16)]
                    for lane in range(16):
                        wv = w16[lane]
                        e = t * 16 + lane
                        for q in range(EMB // 16):
                            sl = pl.ds(q * 16, 16)
                            rows_v[e, sl] = rows_v[e, sl] * wv
                    return c2

                lax.fori_loop(0, ECH // 16, edge16, 0)
                pltpu.sync_copy(rows_v, agg_sh.at[dst_v.at[i]], add=True)
                return c3

            lax.fori_loop(0, 16, chunk, 0)
            return carry

        lax.fori_loop(0, NCH // 16, refill, 0)
        plsc.subcore_barrier()
        for r in range(10):
            sl = pl.ds(s * RPS + r * 64, 64)
            pltpu.sync_copy(agg_sh.at[sl], zbuf)
            pltpu.sync_copy(zbuf, out_hbm.at[c].at[sl])

    return k(x, src3, dst3, w3)


def _tc_mlp(x, agg0, agg1, eps, w1, b1, w2, b2):
    def body(eps_ref, x_ref, a0_ref, a1_ref, w1_ref, b1_ref, w2_ref, b2_ref,
             o_ref):
        out = (2.0 + eps_ref[0]) * x_ref[...] + a0_ref[...] + a1_ref[...]
        h = jnp.dot(out, w1_ref[...], preferred_element_type=jnp.float32)
        h = jnp.maximum(h + b1_ref[...], 0.0)
        h = jnp.dot(h, w2_ref[...], preferred_element_type=jnp.float32)
        o_ref[...] = jnp.maximum(h + b2_ref[...], 0.0)

    blk = 1000
    return pl.pallas_call(
        body,
        grid=(N // blk,),
        in_specs=[
            pl.BlockSpec(memory_space=pltpu.SMEM),
            pl.BlockSpec((blk, HID), lambda i: (i, 0)),
            pl.BlockSpec((blk, HID), lambda i: (i, 0)),
            pl.BlockSpec((blk, HID), lambda i: (i, 0)),
            pl.BlockSpec((HID, HID), lambda i: (0, 0)),
            pl.BlockSpec((1, HID), lambda i: (0, 0)),
            pl.BlockSpec((HID, HID), lambda i: (0, 0)),
            pl.BlockSpec((1, HID), lambda i: (0, 0)),
        ],
        out_specs=pl.BlockSpec((blk, HID), lambda i: (i, 0)),
        out_shape=jax.ShapeDtypeStruct((N, HID), jnp.float32),
    )(eps, x, agg0, agg1, w1, b1, w2, b2)


def _tc_counts(batch2):
    def body(b_ref, starts_ref, counts_ref, mn_ref):
        b = b_ref[...]
        start_g = jnp.int32(0)
        mn = jnp.int32(0)
        for g in range(NG):
            cg = jnp.sum(jnp.where(b == g, 1, 0).astype(jnp.int32))
            counts_ref[g] = cg
            starts_ref[g] = start_g
            start_g = start_g + cg
            mn = jnp.maximum(mn, cg)
        mn_ref[0] = mn

    return pl.pallas_call(
        body,
        in_specs=[pl.BlockSpec((80, 128), lambda: (0, 0))],
        out_specs=[
            pl.BlockSpec(memory_space=pltpu.SMEM),
            pl.BlockSpec(memory_space=pltpu.SMEM),
            pl.BlockSpec(memory_space=pltpu.SMEM),
        ],
        out_shape=[
            jax.ShapeDtypeStruct((NG,), jnp.int32),
            jax.ShapeDtypeStruct((NG,), jnp.int32),
            jax.ShapeDtypeStruct((1,), jnp.int32),
        ],
    )(batch2)


def _tc_conv(xp, starts, counts, mn, w0, b0, w1, b1, w2, b2):
    rows = BLKL + 2 * HALO

    def body(starts_ref, counts_ref, mn_ref, xp_ref,
             w0_ref, b0_ref, w1_ref, b1_ref, w2_ref, b2_ref, o_ref):
        g = pl.program_id(0)
        j = pl.program_id(1)
        mn_v = mn_ref[0]

        @pl.when(jnp.logical_and(g == 0, j == 0))
        def _():
            o_ref[...] = jnp.zeros_like(o_ref)

        @pl.when(j * BLKL < mn_v)
        def _():
            st = starts_ref[g]
            cnt = counts_ref[g]
            base = jnp.minimum(st + j * BLKL, CONV_CLAMP)
            xin = xp_ref[pl.ds(base, rows), :]
            p = (j * BLKL - HALO
                 + lax.broadcasted_iota(jnp.int32, (rows, 1), 0))
            keep_in = jnp.logical_and(p >= 0, p < cnt)
            h = jnp.where(keep_in, xin, 0.0)
            keep_out = jnp.logical_and(p >= 0, p < mn_v)

            for w_ref, b_ref in ((w0_ref, b0_ref), (w1_ref, b1_ref),
                                 (w2_ref, b2_ref)):
                cin = w_ref.shape[1]
                zrow = jnp.zeros((1, cin), jnp.float32)
                hm = jnp.concatenate([zrow, h[:-1, :]], axis=0)
                hp = jnp.concatenate([h[1:, :], zrow], axis=0)
                y = (jnp.dot(hm, w_ref[0], preferred_element_type=jnp.float32)
                     + jnp.dot(h, w_ref[1], preferred_element_type=jnp.float32)
                     + jnp.dot(hp, w_ref[2], preferred_element_type=jnp.float32)
                     + b_ref[...])
                h = jnp.where(keep_out, jnp.maximum(y, 0.0), 0.0)

            o_ref[pl.ds(g, 1), :] += jnp.sum(
                h[HALO:HALO + BLKL, :], axis=0, keepdims=True)

    return pl.pallas_call(
        body,
        grid=(NG, JMAX),
        in_specs=[
            pl.BlockSpec(memory_space=pltpu.SMEM),
            pl.BlockSpec(memory_space=pltpu.SMEM),
            pl.BlockSpec(memory_space=pltpu.SMEM),
            pl.BlockSpec((XP_ROWS, HID), lambda g, j: (0, 0)),
            pl.BlockSpec((3, HID, 64), lambda g, j: (0, 0, 0)),
            pl.BlockSpec((1, 64), lambda g, j: (0, 0)),
            pl.BlockSpec((3, 64, HID), lambda g, j: (0, 0, 0)),
            pl.BlockSpec((1, HID), lambda g, j: (0, 0)),
            pl.BlockSpec((3, HID, 256), lambda g, j: (0, 0, 0)),
            pl.BlockSpec((1, 256), lambda g, j: (0, 0)),
        ],
        out_specs=pl.BlockSpec((NG, 256), lambda g, j: (0, 0)),
        out_shape=jax.ShapeDtypeStruct((NG, 256), jnp.float32),
    )(starts, counts, mn, xp, w0, b0, w1, b1, w2, b2)


def _tc_head(feat, mn, fw1, fb1, fw2p, fb2p):
    def body(mn_ref, f_ref, w1_ref, b1_ref, w2_ref, b2_ref, o_ref):
        inv = 1.0 / mn_ref[0].astype(jnp.float32)
        f = f_ref[...] * inv
        h = jnp.dot(f, w1_ref[...], preferred_element_type=jnp.float32)
        h = jnp.maximum(h + b1_ref[...], 0.0)
        logits = jnp.dot(h, w2_ref[...], preferred_element_type=jnp.float32)
        logits = logits + b2_ref[...]
        lane = lax.broadcasted_iota(jnp.int32, logits.shape, 1)
        logits = jnp.where(lane < 2, logits, -1e30)
        m = jnp.max(logits, axis=1, keepdims=True)
        ex = jnp.exp(logits - m)
        o_ref[...] = ex / jnp.sum(ex, axis=1, keepdims=True)

    return pl.pallas_call(
        body,
        in_specs=[
            pl.BlockSpec(memory_space=pltpu.SMEM),
            pl.BlockSpec((NG, 256), lambda: (0, 0)),
            pl.BlockSpec((256, 128), lambda: (0, 0)),
            pl.BlockSpec((1, 128), lambda: (0, 0)),
            pl.BlockSpec((128, 128), lambda: (0, 0)),
            pl.BlockSpec((1, 128), lambda: (0, 0)),
        ],
        out_specs=pl.BlockSpec((NG, 128), lambda: (0, 0)),
        out_shape=jax.ShapeDtypeStruct((NG, 128), jnp.float32),
    )(mn, feat, fw1, fb1, fw2p, fb2p)


def kernel(x_idx, edge_index, edge_attr, batch, params):
    idx = jnp.concatenate(
        [x_idx.astype(jnp.int32),
         jnp.zeros((B_EMB - N,), jnp.int32)]).reshape(NW, 3, 128)
    src3 = jnp.concatenate(
        [edge_index[0].astype(jnp.int32),
         jnp.zeros((EPAD - E,), jnp.int32)]).reshape(NW, NCH, ECH)
    dst3 = jnp.concatenate(
        [edge_index[1].astype(jnp.int32),
         jnp.zeros((EPAD - E,), jnp.int32)]).reshape(NW, NCH, ECH)
    w3 = jnp.concatenate(
        [edge_attr.astype(jnp.float32),
         jnp.zeros((EPAD - E,), jnp.float32)]).reshape(NW, NCH, ECH)
    batch2 = jnp.concatenate(
        [batch.astype(jnp.int32),
         jnp.full((80 * 128 - N,), NG, jnp.int32)]).reshape(80, 128)

    x = _sc_embed(params['emb'], idx)[:N]

    for l in range(3):
        agg = _sc_msgpass(x, src3, dst3, w3)
        x = _tc_mlp(
            x, agg[0, :N], agg[1, :N], params['eps_%d' % l],
            params['W1_%d' % l], params['b1_%d' % l].reshape(1, HID),
            params['W2_%d' % l], params['b2_%d' % l].reshape(1, HID))

    starts, counts, mn = _tc_counts(batch2)

    xp = jnp.concatenate(
        [jnp.zeros((HALO, HID), jnp.float32),
         x,
         jnp.zeros((XP_ROWS - N - HALO, HID), jnp.float32)])
    cw = [jnp.transpose(params['cw_%d' % c], (2, 1, 0)) for c in range(3)]
    feat = _tc_conv(
        xp, starts, counts, mn,
        cw[0], params['cb_0'].reshape(1, 64),
        cw[1], params['cb_1'].reshape(1, HID),
        cw[2], params['cb_2'].reshape(1, 256))

    fw2p = jnp.zeros((128, 128), jnp.float32).at[:, :2].set(params['fW2'])
    fb2p = jnp.zeros((1, 128), jnp.float32).at[0, :2].set(params['fb2'])
    probs = _tc_head(feat, mn, params['fW1'],
                     params['fb1'].reshape(1, 128), fw2p, fb2p)
    return probs[:, :2]


# speedup vs baseline: 5.2623x; 1.1940x over previous
import functools

import jax
import jax.numpy as jnp
from jax import lax
from jax.experimental import pallas as pl
from jax.experimental.pallas import tpu as pltpu
from jax.experimental.pallas import tpu_sc as plsc

N = 10000
E = 320000
EMB = 128
HID = 128
NG = 16

NW = 32
BPW_EMB = 384
B_EMB = NW * BPW_EMB
ECH = 128
NCH = 80
EPW = ECH * NCH
EPAD = NW * EPW
NP_AGG = 10240
RPS = NP_AGG // 16
BLKL = 512
JMAX = 20
HALO = 8
XP_ROWS = N + 544
CONV_CLAMP = N + HALO


def _sc_embed(emb, idx3):
    mesh = plsc.VectorSubcoreMesh(core_axis_name="c", subcore_axis_name="s")

    @functools.partial(
        pl.kernel,
        mesh=mesh,
        out_type=jax.ShapeDtypeStruct((B_EMB, EMB), jnp.float32),
        scratch_types=[
            pltpu.VMEM((3, 128), jnp.int32),
            pltpu.VMEM((BPW_EMB, EMB), jnp.float32),
            pltpu.SemaphoreType.DMA,
        ],
    )
    def k(emb_hbm, idx_hbm, out_hbm, idx_v, rows_v, sem):
        wid = lax.axis_index("s") * 2 + lax.axis_index("c")
        pltpu.sync_copy(idx_hbm.at[wid], idx_v)
        for ch in range(3):
            pltpu.async_copy(
                emb_hbm.at[idx_v.at[ch]],
                rows_v.at[pl.ds(ch * 128, 128)],
                sem,
            )
        for ch in range(3):
            pltpu.make_async_copy(
                emb_hbm.at[idx_v.at[ch]],
                rows_v.at[pl.ds(ch * 128, 128)],
                sem,
            ).wait()
        pltpu.sync_copy(rows_v, out_hbm.at[pl.ds(wid * BPW_EMB, BPW_EMB)])

    return k(emb, idx3)


def _sc_msgpass(x, src3, dst3, w3):
    mesh = plsc.VectorSubcoreMesh(core_axis_name="c", subcore_axis_name="s")

    @functools.partial(
        pl.kernel,
        mesh=mesh,
        out_type=jax.ShapeDtypeStruct((2, NP_AGG, EMB), jnp.float32),
        scratch_types=[
            pltpu.VMEM((16, ECH), jnp.int32),
            pltpu.VMEM((16, ECH), jnp.int32),
            pltpu.VMEM((16, ECH), jnp.float32),
            pltpu.VMEM((ECH, EMB), jnp.float32),
            pltpu.VMEM((ECH, EMB), jnp.float32),
            pltpu.VMEM((64, EMB), jnp.float32),
            pltpu.VMEM_SHARED((NP_AGG, EMB), jnp.float32),
            pltpu.SemaphoreType.DMA,
            pltpu.SemaphoreType.DMA,
        ],
    )
    def k(x_hbm, src_hbm, dst_hbm, w_hbm, out_hbm,
          src_v, dst_v, w_v, rows_a, rows_b, zbuf, agg_sh, sem_a, sem_b):
        c = lax.axis_index("c")
        s = lax.axis_index("s")
        wid = s * 2 + c

        def zrow(i, carry):
            for q in range(EMB // 16):
                zbuf[i, pl.ds(q * 16, 16)] = jnp.zeros((16,), jnp.float32)
            return carry

        lax.fori_loop(0, 64, zrow, 0)
        for r in range(10):
            pltpu.sync_copy(zbuf, agg_sh.at[pl.ds(s * RPS + r * 64, 64)])
        plsc.subcore_barrier()

        def fire(i, buf, sem):
            pltpu.async_copy(x_hbm.at[src_v.at[i]], buf, sem)

        def drain(i, buf, sem):
            pltpu.make_async_copy(x_hbm.at[src_v.at[i]], buf, sem).wait()

        def scale_scatter(i, buf):
            def edge16(t, c2):
                w16 = w_v[i, pl.ds(t * 16, 16)]
                for lane in range(16):
                    wv = w16[lane]
                    e = t * 16 + lane
                    for q in range(EMB // 16):
                        sl = pl.ds(q * 16, 16)
                        buf[e, sl] = buf[e, sl] * wv
                return c2

            lax.fori_loop(0, ECH // 16, edge16, 0)
            pltpu.sync_copy(buf, agg_sh.at[dst_v.at[i]], add=True)

        def refill(rb, carry):
            wsl = pl.ds(rb * 16, 16)
            pltpu.sync_copy(src_hbm.at[wid].at[wsl], src_v)
            pltpu.sync_copy(dst_hbm.at[wid].at[wsl], dst_v)
            pltpu.sync_copy(w_hbm.at[wid].at[wsl], w_v)
            fire(0, rows_a, sem_a)

            def pair(ip, c3):
                i0 = ip * 2
                i1 = i0 + 1
                fire(i1, rows_b, sem_b)
                drain(i0, rows_a, sem_a)
                scale_scatter(i0, rows_a)

                @pl.when(ip < 7)
                def _():
                    fire(i0 + 2, rows_a, sem_a)

                drain(i1, rows_b, sem_b)
                scale_scatter(i1, rows_b)
                return c3

            lax.fori_loop(0, 8, pair, 0)
            return carry

        lax.fori_loop(0, NCH // 16, refill, 0)
        plsc.subcore_barrier()
        for r in range(10):
            sl = pl.ds(s * RPS + r * 64, 64)
            pltpu.sync_copy(agg_sh.at[sl], zbuf)
            pltpu.sync_copy(zbuf, out_hbm.at[c].at[sl])

    return k(x, src3, dst3, w3)


def _tc_mlp(x, agg0, agg1, eps, w1, b1, w2, b2):
    def body(eps_ref, x_ref, a0_ref, a1_ref, w1_ref, b1_ref, w2_ref, b2_ref,
             o_ref):
        out = (2.0 + eps_ref[0]) * x_ref[...] + a0_ref[...] + a1_ref[...]
        h = jnp.dot(out, w1_ref[...], preferred_element_type=jnp.float32)
        h = jnp.maximum(h + b1_ref[...], 0.0)
        h = jnp.dot(h, w2_ref[...], preferred_element_type=jnp.float32)
        o_ref[...] = jnp.maximum(h + b2_ref[...], 0.0)

    blk = 1000
    return pl.pallas_call(
        body,
        grid=(N // blk,),
        in_specs=[
            pl.BlockSpec(memory_space=pltpu.SMEM),
            pl.BlockSpec((blk, HID), lambda i: (i, 0)),
            pl.BlockSpec((blk, HID), lambda i: (i, 0)),
            pl.BlockSpec((blk, HID), lambda i: (i, 0)),
            pl.BlockSpec((HID, HID), lambda i: (0, 0)),
            pl.BlockSpec((1, HID), lambda i: (0, 0)),
            pl.BlockSpec((HID, HID), lambda i: (0, 0)),
            pl.BlockSpec((1, HID), lambda i: (0, 0)),
        ],
        out_specs=pl.BlockSpec((blk, HID), lambda i: (i, 0)),
        out_shape=jax.ShapeDtypeStruct((N, HID), jnp.float32),
    )(eps, x, agg0, agg1, w1, b1, w2, b2)


def _tc_counts(batch2):
    def body(b_ref, starts_ref, counts_ref, mn_ref):
        b = b_ref[...]
        start_g = jnp.int32(0)
        mn = jnp.int32(0)
        for g in range(NG):
            cg = jnp.sum(jnp.where(b == g, 1, 0).astype(jnp.int32))
            counts_ref[g] = cg
            starts_ref[g] = start_g
            start_g = start_g + cg
            mn = jnp.maximum(mn, cg)
        mn_ref[0] = mn

    return pl.pallas_call(
        body,
        in_specs=[pl.BlockSpec((80, 128), lambda: (0, 0))],
        out_specs=[
            pl.BlockSpec(memory_space=pltpu.SMEM),
            pl.BlockSpec(memory_space=pltpu.SMEM),
            pl.BlockSpec(memory_space=pltpu.SMEM),
        ],
        out_shape=[
            jax.ShapeDtypeStruct((NG,), jnp.int32),
            jax.ShapeDtypeStruct((NG,), jnp.int32),
            jax.ShapeDtypeStruct((1,), jnp.int32),
        ],
    )(batch2)


def _tc_conv(xp, starts, counts, mn, w0, b0, w1, b1, w2, b2):
    rows = BLKL + 2 * HALO

    def body(starts_ref, counts_ref, mn_ref, xp_ref,
             w0_ref, b0_ref, w1_ref, b1_ref, w2_ref, b2_ref, o_ref):
        g = pl.program_id(0)
        j = pl.program_id(1)
        mn_v = mn_ref[0]

        @pl.when(jnp.logical_and(g == 0, j == 0))
        def _():
            o_ref[...] = jnp.zeros_like(o_ref)

        @pl.when(j * BLKL < mn_v)
        def _():
            st = starts_ref[g]
            cnt = counts_ref[g]
            base = jnp.minimum(st + j * BLKL, CONV_CLAMP)
            xin = xp_ref[pl.ds(base, rows), :]
            p = (j * BLKL - HALO
                 + lax.broadcasted_iota(jnp.int32, (rows, 1), 0))
            keep_in = jnp.logical_and(p >= 0, p < cnt)
            h = jnp.where(keep_in, xin, 0.0)
            keep_out = jnp.logical_and(p >= 0, p < mn_v)

            for w_ref, b_ref in ((w0_ref, b0_ref), (w1_ref, b1_ref),
                                 (w2_ref, b2_ref)):
                cin = w_ref.shape[1]
                zrow = jnp.zeros((1, cin), jnp.float32)
                hm = jnp.concatenate([zrow, h[:-1, :]], axis=0)
                hp = jnp.concatenate([h[1:, :], zrow], axis=0)
                y = (jnp.dot(hm, w_ref[0], preferred_element_type=jnp.float32)
                     + jnp.dot(h, w_ref[1], preferred_element_type=jnp.float32)
                     + jnp.dot(hp, w_ref[2], preferred_element_type=jnp.float32)
                     + b_ref[...])
                h = jnp.where(keep_out, jnp.maximum(y, 0.0), 0.0)

            o_ref[pl.ds(g, 1), :] += jnp.sum(
                h[HALO:HALO + BLKL, :], axis=0, keepdims=True)

    return pl.pallas_call(
        body,
        grid=(NG, JMAX),
        in_specs=[
            pl.BlockSpec(memory_space=pltpu.SMEM),
            pl.BlockSpec(memory_space=pltpu.SMEM),
            pl.BlockSpec(memory_space=pltpu.SMEM),
            pl.BlockSpec((XP_ROWS, HID), lambda g, j: (0, 0)),
            pl.BlockSpec((3, HID, 64), lambda g, j: (0, 0, 0)),
            pl.BlockSpec((1, 64), lambda g, j: (0, 0)),
            pl.BlockSpec((3, 64, HID), lambda g, j: (0, 0, 0)),
            pl.BlockSpec((1, HID), lambda g, j: (0, 0)),
            pl.BlockSpec((3, HID, 256), lambda g, j: (0, 0, 0)),
            pl.BlockSpec((1, 256), lambda g, j: (0, 0)),
        ],
        out_specs=pl.BlockSpec((NG, 256), lambda g, j: (0, 0)),
        out_shape=jax.ShapeDtypeStruct((NG, 256), jnp.float32),
    )(starts, counts, mn, xp, w0, b0, w1, b1, w2, b2)


def _tc_head(feat, mn, fw1, fb1, fw2p, fb2p):
    def body(mn_ref, f_ref, w1_ref, b1_ref, w2_ref, b2_ref, o_ref):
        inv = 1.0 / mn_ref[0].astype(jnp.float32)
        f = f_ref[...] * inv
        h = jnp.dot(f, w1_ref[...], preferred_element_type=jnp.float32)
        h = jnp.maximum(h + b1_ref[...], 0.0)
        logits = jnp.dot(h, w2_ref[...], preferred_element_type=jnp.float32)
        logits = logits + b2_ref[...]
        lane = lax.broadcasted_iota(jnp.int32, logits.shape, 1)
        logits = jnp.where(lane < 2, logits, -1e30)
        m = jnp.max(logits, axis=1, keepdims=True)
        ex = jnp.exp(logits - m)
        o_ref[...] = ex / jnp.sum(ex, axis=1, keepdims=True)

    return pl.pallas_call(
        body,
        in_specs=[
            pl.BlockSpec(memory_space=pltpu.SMEM),
            pl.BlockSpec((NG, 256), lambda: (0, 0)),
            pl.BlockSpec((256, 128), lambda: (0, 0)),
            pl.BlockSpec((1, 128), lambda: (0, 0)),
            pl.BlockSpec((128, 128), lambda: (0, 0)),
            pl.BlockSpec((1, 128), lambda: (0, 0)),
        ],
        out_specs=pl.BlockSpec((NG, 128), lambda: (0, 0)),
        out_shape=jax.ShapeDtypeStruct((NG, 128), jnp.float32),
    )(mn, feat, fw1, fb1, fw2p, fb2p)


def kernel(x_idx, edge_index, edge_attr, batch, params):
    idx = jnp.concatenate(
        [x_idx.astype(jnp.int32),
         jnp.zeros((B_EMB - N,), jnp.int32)]).reshape(NW, 3, 128)
    src3 = jnp.concatenate(
        [edge_index[0].astype(jnp.int32),
         jnp.zeros((EPAD - E,), jnp.int32)]).reshape(NW, NCH, ECH)
    dst3 = jnp.concatenate(
        [edge_index[1].astype(jnp.int32),
         jnp.zeros((EPAD - E,), jnp.int32)]).reshape(NW, NCH, ECH)
    w3 = jnp.concatenate(
        [edge_attr.astype(jnp.float32),
         jnp.zeros((EPAD - E,), jnp.float32)]).reshape(NW, NCH, ECH)
    batch2 = jnp.concatenate(
        [batch.astype(jnp.int32),
         jnp.full((80 * 128 - N,), NG, jnp.int32)]).reshape(80, 128)

    x = _sc_embed(params['emb'], idx)[:N]

    for l in range(3):
        agg = _sc_msgpass(x, src3, dst3, w3)
        x = _tc_mlp(
            x, agg[0, :N], agg[1, :N], params['eps_%d' % l],
            params['W1_%d' % l], params['b1_%d' % l].reshape(1, HID),
            params['W2_%d' % l], params['b2_%d' % l].reshape(1, HID))

    starts, counts, mn = _tc_counts(batch2)

    xp = jnp.concatenate(
        [jnp.zeros((HALO, HID), jnp.float32),
         x,
         jnp.zeros((XP_ROWS - N - HALO, HID), jnp.float32)])
    cw = [jnp.transpose(params['cw_%d' % c], (2, 1, 0)) for c in range(3)]
    feat = _tc_conv(
        xp, starts, counts, mn,
        cw[0], params['cb_0'].reshape(1, 64),
        cw[1], params['cb_1'].reshape(1, HID),
        cw[2], params['cb_2'].reshape(1, 256))

    fw2p = jnp.zeros((128, 128), jnp.float32).at[:, :2].set(params['fW2'])
    fb2p = jnp.zeros((1, 128), jnp.float32).at[0, :2].set(params['fb2'])
    probs = _tc_head(feat, mn, params['fW1'],
                     params['fb1'].reshape(1, 128), fw2p, fb2p)
    return probs[:, :2]


# speedup vs baseline: 5.2842x; 1.0042x over previous
import functools

import jax
import jax.numpy as jnp
from jax import lax
from jax.experimental import pallas as pl
from jax.experimental.pallas import tpu as pltpu
from jax.experimental.pallas import tpu_sc as plsc

N = 10000
E = 320000
EMB = 128
HID = 128
NG = 16

NW = 32
BPW_EMB = 384
B_EMB = NW * BPW_EMB
ECH = 128
NCHT = 2560
EPAD = NCHT * ECH
W_C1 = 7
W_C0 = 3
C1_CHUNKS = W_C1 * 16
C0_CHUNKS = W_C0 * 16
C0_BASE = 16 * C1_CHUNKS
NP_AGG = 10240
RPS = NP_AGG // 16
BLKL = 512
JMAX = 20
HALO = 8
XP_ROWS = N + 544
CONV_CLAMP = N + HALO


def _sc_embed(emb, idx3):
    mesh = plsc.VectorSubcoreMesh(core_axis_name="c", subcore_axis_name="s")

    @functools.partial(
        pl.kernel,
        mesh=mesh,
        out_type=jax.ShapeDtypeStruct((B_EMB, EMB), jnp.float32),
        scratch_types=[
            pltpu.VMEM((3, 128), jnp.int32),
            pltpu.VMEM((BPW_EMB, EMB), jnp.float32),
            pltpu.SemaphoreType.DMA,
        ],
    )
    def k(emb_hbm, idx_hbm, out_hbm, idx_v, rows_v, sem):
        wid = lax.axis_index("s") * 2 + lax.axis_index("c")
        pltpu.sync_copy(idx_hbm.at[wid], idx_v)
        for ch in range(3):
            pltpu.async_copy(
                emb_hbm.at[idx_v.at[ch]],
                rows_v.at[pl.ds(ch * 128, 128)],
                sem,
            )
        for ch in range(3):
            pltpu.make_async_copy(
                emb_hbm.at[idx_v.at[ch]],
                rows_v.at[pl.ds(ch * 128, 128)],
                sem,
            ).wait()
        pltpu.sync_copy(rows_v, out_hbm.at[pl.ds(wid * BPW_EMB, BPW_EMB)])

    return k(emb, idx3)


def _sc_msgpass(x, src3, dst3, w3):
    mesh = plsc.VectorSubcoreMesh(core_axis_name="c", subcore_axis_name="s")

    @functools.partial(
        pl.kernel,
        mesh=mesh,
        out_type=jax.ShapeDtypeStruct((2, NP_AGG, EMB), jnp.float32),
        scratch_types=[
            pltpu.VMEM((16, ECH), jnp.int32),
            pltpu.VMEM((16, ECH), jnp.int32),
            pltpu.VMEM((16, ECH), jnp.float32),
            pltpu.VMEM((ECH, EMB), jnp.float32),
            pltpu.VMEM((ECH, EMB), jnp.float32),
            pltpu.VMEM((64, EMB), jnp.float32),
            pltpu.VMEM_SHARED((NP_AGG, EMB), jnp.float32),
            pltpu.SemaphoreType.DMA,
            pltpu.SemaphoreType.DMA,
        ],
    )
    def k(x_hbm, src_hbm, dst_hbm, w_hbm, out_hbm,
          src_v, dst_v, w_v, rows_a, rows_b, zbuf, agg_sh, sem_a, sem_b):
        c = lax.axis_index("c")
        s = lax.axis_index("s")
        nwin = jnp.where(c == 1, W_C1, W_C0)
        cbase = jnp.where(c == 1, s * C1_CHUNKS, C0_BASE + s * C0_CHUNKS)

        def zrow(i, carry):
            for q in range(EMB // 16):
                zbuf[i, pl.ds(q * 16, 16)] = jnp.zeros((16,), jnp.float32)
            return carry

        lax.fori_loop(0, 64, zrow, 0)
        for r in range(10):
            pltpu.sync_copy(zbuf, agg_sh.at[pl.ds(s * RPS + r * 64, 64)])
        plsc.subcore_barrier()

        def fire(i, buf, sem):
            pltpu.async_copy(x_hbm.at[src_v.at[i]], buf, sem)

        def drain(i, buf, sem):
            pltpu.make_async_copy(x_hbm.at[src_v.at[i]], buf, sem).wait()

        def scale_scatter(i, buf):
            def edge16(t, c2):
                w16 = w_v[i, pl.ds(t * 16, 16)]
                for lane in range(16):
                    wv = w16[lane]
                    e = t * 16 + lane
                    for q in range(EMB // 16):
                        sl = pl.ds(q * 16, 16)
                        buf[e, sl] = buf[e, sl] * wv
                return c2

            lax.fori_loop(0, ECH // 16, edge16, 0)
            pltpu.sync_copy(buf, agg_sh.at[dst_v.at[i]], add=True)

        def refill(rb, carry):
            wsl = pl.ds(cbase + rb * 16, 16)
            pltpu.sync_copy(src_hbm.at[wsl], src_v)
            pltpu.sync_copy(dst_hbm.at[wsl], dst_v)
            pltpu.sync_copy(w_hbm.at[wsl], w_v)
            fire(0, rows_a, sem_a)

            def pair(ip, c3):
                i0 = ip * 2
                i1 = i0 + 1
                fire(i1, rows_b, sem_b)
                drain(i0, rows_a, sem_a)
                scale_scatter(i0, rows_a)

                @pl.when(ip < 7)
                def _():
                    fire(i0 + 2, rows_a, sem_a)

                drain(i1, rows_b, sem_b)
                scale_scatter(i1, rows_b)
                return c3

            lax.fori_loop(0, 8, pair, 0)
            return carry

        lax.fori_loop(0, nwin, refill, 0)
        plsc.subcore_barrier()
        for r in range(10):
            sl = pl.ds(s * RPS + r * 64, 64)
            pltpu.sync_copy(agg_sh.at[sl], zbuf)
            pltpu.sync_copy(zbuf, out_hbm.at[c].at[sl])

    return k(x, src3, dst3, w3)


def _tc_mlp(x, agg0, agg1, eps, w1, b1, w2, b2):
    def body(eps_ref, x_ref, a0_ref, a1_ref, w1_ref, b1_ref, w2_ref, b2_ref,
             o_ref):
        out = (2.0 + eps_ref[0]) * x_ref[...] + a0_ref[...] + a1_ref[...]
        h = jnp.dot(out, w1_ref[...], preferred_element_type=jnp.float32)
        h = jnp.maximum(h + b1_ref[...], 0.0)
        h = jnp.dot(h, w2_ref[...], preferred_element_type=jnp.float32)
        o_ref[...] = jnp.maximum(h + b2_ref[...], 0.0)

    blk = 1000
    return pl.pallas_call(
        body,
        grid=(N // blk,),
        in_specs=[
            pl.BlockSpec(memory_space=pltpu.SMEM),
            pl.BlockSpec((blk, HID), lambda i: (i, 0)),
            pl.BlockSpec((blk, HID), lambda i: (i, 0)),
            pl.BlockSpec((blk, HID), lambda i: (i, 0)),
            pl.BlockSpec((HID, HID), lambda i: (0, 0)),
            pl.BlockSpec((1, HID), lambda i: (0, 0)),
            pl.BlockSpec((HID, HID), lambda i: (0, 0)),
            pl.BlockSpec((1, HID), lambda i: (0, 0)),
        ],
        out_specs=pl.BlockSpec((blk, HID), lambda i: (i, 0)),
        out_shape=jax.ShapeDtypeStruct((N, HID), jnp.float32),
    )(eps, x, agg0, agg1, w1, b1, w2, b2)


def _tc_counts(batch2):
    def body(b_ref, starts_ref, counts_ref, mn_ref):
        b = b_ref[...]
        start_g = jnp.int32(0)
        mn = jnp.int32(0)
        for g in range(NG):
            cg = jnp.sum(jnp.where(b == g, 1, 0).astype(jnp.int32))
            counts_ref[g] = cg
            starts_ref[g] = start_g
            start_g = start_g + cg
            mn = jnp.maximum(mn, cg)
        mn_ref[0] = mn

    return pl.pallas_call(
        body,
        in_specs=[pl.BlockSpec((80, 128), lambda: (0, 0))],
        out_specs=[
            pl.BlockSpec(memory_space=pltpu.SMEM),
            pl.BlockSpec(memory_space=pltpu.SMEM),
            pl.BlockSpec(memory_space=pltpu.SMEM),
        ],
        out_shape=[
            jax.ShapeDtypeStruct((NG,), jnp.int32),
            jax.ShapeDtypeStruct((NG,), jnp.int32),
            jax.ShapeDtypeStruct((1,), jnp.int32),
        ],
    )(batch2)


def _tc_conv(xp, starts, counts, mn, w0, b0, w1, b1, w2, b2):
    rows = BLKL + 2 * HALO

    def body(starts_ref, counts_ref, mn_ref, xp_ref,
             w0_ref, b0_ref, w1_ref, b1_ref, w2_ref, b2_ref, o_ref):
        g = pl.program_id(0)
        j = pl.program_id(1)
        mn_v = mn_ref[0]

        @pl.when(jnp.logical_and(g == 0, j == 0))
        def _():
            o_ref[...] = jnp.zeros_like(o_ref)

        @pl.when(j * BLKL < mn_v)
        def _():
            st = starts_ref[g]
            cnt = counts_ref[g]
            base = jnp.minimum(st + j * BLKL, CONV_CLAMP)
            xin = xp_ref[pl.ds(base, rows), :]
            p = (j * BLKL - HALO
                 + lax.broadcasted_iota(jnp.int32, (rows, 1), 0))
            keep_in = jnp.logical_and(p >= 0, p < cnt)
            h = jnp.where(keep_in, xin, 0.0)
            keep_out = jnp.logical_and(p >= 0, p < mn_v)

            for w_ref, b_ref in ((w0_ref, b0_ref), (w1_ref, b1_ref),
                                 (w2_ref, b2_ref)):
                cin = w_ref.shape[1]
                zrow = jnp.zeros((1, cin), jnp.float32)
                hm = jnp.concatenate([zrow, h[:-1, :]], axis=0)
                hp = jnp.concatenate([h[1:, :], zrow], axis=0)
                y = (jnp.dot(hm, w_ref[0], preferred_element_type=jnp.float32)
                     + jnp.dot(h, w_ref[1], preferred_element_type=jnp.float32)
                     + jnp.dot(hp, w_ref[2], preferred_element_type=jnp.float32)
                     + b_ref[...])
                h = jnp.where(keep_out, jnp.maximum(y, 0.0), 0.0)

            o_ref[pl.ds(g, 1), :] += jnp.sum(
                h[HALO:HALO + BLKL, :], axis=0, keepdims=True)

    return pl.pallas_call(
        body,
        grid=(NG, JMAX),
        in_specs=[
            pl.BlockSpec(memory_space=pltpu.SMEM),
            pl.BlockSpec(memory_space=pltpu.SMEM),
            pl.BlockSpec(memory_space=pltpu.SMEM),
            pl.BlockSpec((XP_ROWS, HID), lambda g, j: (0, 0)),
            pl.BlockSpec((3, HID, 64), lambda g, j: (0, 0, 0)),
            pl.BlockSpec((1, 64), lambda g, j: (0, 0)),
            pl.BlockSpec((3, 64, HID), lambda g, j: (0, 0, 0)),
            pl.BlockSpec((1, HID), lambda g, j: (0, 0)),
            pl.BlockSpec((3, HID, 256), lambda g, j: (0, 0, 0)),
            pl.BlockSpec((1, 256), lambda g, j: (0, 0)),
        ],
        out_specs=pl.BlockSpec((NG, 256), lambda g, j: (0, 0)),
        out_shape=jax.ShapeDtypeStruct((NG, 256), jnp.float32),
    )(starts, counts, mn, xp, w0, b0, w1, b1, w2, b2)


def _tc_head(feat, mn, fw1, fb1, fw2p, fb2p):
    def body(mn_ref, f_ref, w1_ref, b1_ref, w2_ref, b2_ref, o_ref):
        inv = 1.0 / mn_ref[0].astype(jnp.float32)
        f = f_ref[...] * inv
        h = jnp.dot(f, w1_ref[...], preferred_element_type=jnp.float32)
        h = jnp.maximum(h + b1_ref[...], 0.0)
        logits = jnp.dot(h, w2_ref[...], preferred_element_type=jnp.float32)
        logits = logits + b2_ref[...]
        lane = lax.broadcasted_iota(jnp.int32, logits.shape, 1)
        logits = jnp.where(lane < 2, logits, -1e30)
        m = jnp.max(logits, axis=1, keepdims=True)
        ex = jnp.exp(logits - m)
        o_ref[...] = ex / jnp.sum(ex, axis=1, keepdims=True)

    return pl.pallas_call(
        body,
        in_specs=[
            pl.BlockSpec(memory_space=pltpu.SMEM),
            pl.BlockSpec((NG, 256), lambda: (0, 0)),
            pl.BlockSpec((256, 128), lambda: (0, 0)),
            pl.BlockSpec((1, 128), lambda: (0, 0)),
            pl.BlockSpec((128, 128), lambda: (0, 0)),
            pl.BlockSpec((1, 128), lambda: (0, 0)),
        ],
        out_specs=pl.BlockSpec((NG, 128), lambda: (0, 0)),
        out_shape=jax.ShapeDtypeStruct((NG, 128), jnp.float32),
    )(mn, feat, fw1, fb1, fw2p, fb2p)


def kernel(x_idx, edge_index, edge_attr, batch, params):
    idx = jnp.concatenate(
        [x_idx.astype(jnp.int32),
         jnp.zeros((B_EMB - N,), jnp.int32)]).reshape(NW, 3, 128)
    src3 = jnp.concatenate(
        [edge_index[0].astype(jnp.int32),
         jnp.zeros((EPAD - E,), jnp.int32)]).reshape(NCHT, ECH)
    dst3 = jnp.concatenate(
        [edge_index[1].astype(jnp.int32),
         jnp.zeros((EPAD - E,), jnp.int32)]).reshape(NCHT, ECH)
    w3 = jnp.concatenate(
        [edge_attr.astype(jnp.float32),
         jnp.zeros((EPAD - E,), jnp.float32)]).reshape(NCHT, ECH)
    batch2 = jnp.concatenate(
        [batch.astype(jnp.int32),
         jnp.full((80 * 128 - N,), NG, jnp.int32)]).reshape(80, 128)

    x = _sc_embed(params['emb'], idx)[:N]

    for l in range(3):
        agg = _sc_msgpass(x, src3, dst3, w3)
        x = _tc_mlp(
            x, agg[0, :N], agg[1, :N], params['eps_%d' % l],
            params['W1_%d' % l], params['b1_%d' % l].reshape(1, HID),
            params['W2_%d' % l], params['b2_%d' % l].reshape(1, HID))

    starts, counts, mn = _tc_counts(batch2)

    xp = jnp.concatenate(
        [jnp.zeros((HALO, HID), jnp.float32),
         x,
         jnp.zeros((XP_ROWS - N - HALO, HID), jnp.float32)])
    cw = [jnp.transpose(params['cw_%d' % c], (2, 1, 0)) for c in range(3)]
    feat = _tc_conv(
        xp, starts, counts, mn,
        cw[0], params['cb_0'].reshape(1, 64),
        cw[1], params['cb_1'].reshape(1, HID),
        cw[2], params['cb_2'].reshape(1, 256))

    fw2p = jnp.zeros((128, 128), jnp.float32).at[:, :2].set(params['fW2'])
    fb2p = jnp.zeros((1, 128), jnp.float32).at[0, :2].set(params['fb2'])
    probs = _tc_head(feat, mn, params['fW1'],
                     params['fb1'].reshape(1, 128), fw2p, fb2p)
    return probs[:, :2]
